# baseline (speedup 1.0000x reference)
import jax
import jax.numpy as jnp
import numpy as np
from jax import lax
from jax.experimental import pallas as pl
from jax.experimental.pallas import tpu as pltpu

_VOCAB = 4096
_VT = 1024
_NV = _VOCAB // _VT
_C = 32
_B = 64
_HW = 16
_P_FULL = _HW * _HW
_PATCH = (1, 2, 3, 4, 5, 6, 8, 10, 13, 16)
_SN = len(_PATCH)
_BETA = 0.25
_RESR = 0.5
_NSHARE = 4

_TICKS = np.linspace(1.0 / 3.0 / _NSHARE, 1.0 - 1.0 / 3.0 / _NSHARE, _NSHARE)
_KSEL = [int(np.argmin(np.abs(_TICKS - i / (_SN - 1)))) for i in range(_SN)]

_PPAD = {pn: ((pn * pn + 7) // 8) * 8 for pn in _PATCH}


def _area_mat(in_size, out_size):
    W = np.zeros((out_size, in_size), dtype=np.float32)
    for i in range(out_size):
        s = (i * in_size) // out_size
        e = -((-(i + 1) * in_size) // out_size)
        W[i, s:e] = 1.0 / (e - s)
    return W


def _cubic_w(t, a=-0.75):
    at = abs(t)
    if at <= 1.0:
        return (a + 2.0) * at ** 3 - (a + 3.0) * at ** 2 + 1.0
    elif at < 2.0:
        return a * at ** 3 - 5.0 * a * at ** 2 + 8.0 * a * at - 4.0 * a
    return 0.0


def _bicubic_mat(in_size, out_size):
    W = np.zeros((out_size, in_size), dtype=np.float64)
    scale = in_size / out_size
    for i in range(out_size):
        x = (i + 0.5) * scale - 0.5
        x0 = int(np.floor(x))
        for t in range(x0 - 1, x0 + 3):
            w = _cubic_w(x - t)
            W[i, min(max(t, 0), in_size - 1)] += w
    return W.astype(np.float32)


import ml_dtypes


def _rb(a):
    return np.asarray(a, dtype=ml_dtypes.bfloat16).astype(np.float32)


def _down_mats(pn):
    M = _rb(_area_mat(_HW, pn))
    A = np.zeros((_P_FULL, _HW * pn), dtype=np.float32)
    for h in range(_HW):
        A[h * _HW:(h + 1) * _HW, h * pn:(h + 1) * pn] = M.T
    Bm = np.zeros((_HW * pn, _PPAD[pn]), dtype=np.float32)
    for p in range(pn):
        Bm[p::pn, p:pn * pn:pn] = M.T
    return A, Bm


def _up_mats(pn):
    M = _rb(_bicubic_mat(pn, _HW))
    A = np.zeros((_PPAD[pn], pn * _HW), dtype=np.float32)
    for h in range(pn):
        A[h * pn:(h + 1) * pn, h * _HW:(h + 1) * _HW] = M.T
    Bm = np.zeros((pn * _HW, _P_FULL), dtype=np.float32)
    for p in range(_HW):
        Bm[p::_HW, p::_HW] = M.T
    return A, Bm


def _kron1_down():
    M = _rb(_area_mat(_HW, 1))
    K = np.outer(M[0], M[0]).reshape(1, _P_FULL)
    Kp = np.zeros((_P_FULL, _PPAD[1]), dtype=np.float32)
    Kp[:, 0] = _rb(K[0])
    return Kp


def _kron1_up():
    M = _rb(_bicubic_mat(1, _HW))
    K = np.outer(M[:, 0], M[:, 0]).reshape(_P_FULL)
    Up = np.zeros((_PPAD[1], _P_FULL), dtype=np.float32)
    Up[0, :] = _rb(K)
    return Up


_DOWN_A = {}
_DOWN_B = {}
_UP_A = {}
_UP_B = {}
for _pn in _PATCH[1:-1]:
    _DOWN_A[_pn], _DOWN_B[_pn] = _down_mats(_pn)
    _UP_A[_pn], _UP_B[_pn] = _up_mats(_pn)
_DOWN1 = _kron1_down()
_UP1 = _kron1_up()

_XMASK = np.ones((8, _P_FULL), dtype=np.float32)
_XMASK[0, 0::16] = 0.0
_XMASK[1, 15::16] = 0.0

_F32 = jnp.float32
_HI = lax.Precision.HIGHEST


def _dot(a, b, dims):
    return lax.dot_general(a, b, (dims, ((), ())), precision=_HI,
                           preferred_element_type=_F32)


_BF16 = jnp.bfloat16


def _dotb(a, b, dims=((1,), (0,))):
    return lax.dot_general(a.astype(_BF16), b.astype(_BF16),
                           (dims, ((), ())), preferred_element_type=_F32)


def _quantize(rows, emb, hen):
    n = rows.shape[0]
    best = None
    bidx = None
    for v in range(_NV):
        ev = emb[v * _VT:(v + 1) * _VT, :]
        s = _dotb(rows, ev, ((1,), (1,))) - hen[:, v * _VT:(v + 1) * _VT]
        m = jnp.max(s, axis=1, keepdims=True)
        iota = lax.broadcasted_iota(jnp.int32, (n, _VT), 1)
        iv = jnp.min(jnp.where(s == m, iota, _VT), axis=1, keepdims=True)
        iv = iv + v * _VT
        if v == 0:
            best, bidx = m, iv
        else:
            bidx = jnp.where(m > best, iv, bidx)
            best = jnp.maximum(m, best)
    h = jnp.zeros((n, _C), _F32)
    for v in range(_NV):
        ev = emb[v * _VT:(v + 1) * _VT, :]
        iota = lax.broadcasted_iota(jnp.int32, (n, _VT), 1)
        oh = (iota == bidx - v * _VT).astype(_F32)
        h = h + _dot(oh, ev, ((1,), (0,)))
    return h


def _shift_lanes(x, o):
    if o == 0:
        return x
    z = jnp.zeros((_C, abs(o)), _F32)
    if o > 0:
        return jnp.concatenate([x[:, o:], z], axis=1)
    return jnp.concatenate([z, x[:, :o]], axis=1)


def _conv3x3(hup, w, b, xmask):
    acc = None
    for ky in range(3):
        for kx in range(3):
            o = 16 * (ky - 1) + (kx - 1)
            sh = _shift_lanes(hup, o)
            if kx == 0:
                sh = sh * xmask[0:1, :]
            elif kx == 2:
                sh = sh * xmask[1:2, :]
            t = _dotb(w[ky, kx], sh)
            acc = t if acc is None else acc + t
    return acc + b


def _make_body(s, pn, last):
    pp = _PPAD[pn]

    def body(*refs):
        if last:
            (fres_ref, f_ref, emb_ref, hen_ref, w_ref, b_ref, xm_ref,
             out_ref, loss_ref, fhat_ref) = refs
        elif s == 0:
            (fres_ref, emb_ref, hen_ref, w_ref, b_ref, xm_ref,
             down_ref, up_ref, out_ref, loss_ref) = refs
        else:
            (fres_ref, emb_ref, hen_ref, w_ref, b_ref, xm_ref,
             downa_ref, downb_ref, upa_ref, upb_ref,
             out_ref, loss_ref) = refs
        chunk = fres_ref[:]
        emb = emb_ref[:]
        hen = hen_ref[:]

        if last:
            rows = jnp.transpose(chunk, (1, 0))
        elif s == 0:
            res2 = _dotb(chunk, down_ref[:])
            rows = jnp.transpose(res2, (1, 0))
        else:
            t1 = _dotb(chunk, downa_ref[:])
            res2 = _dotb(t1, downb_ref[:])
            rows = jnp.transpose(res2, (1, 0))

        h = _quantize(rows, emb, hen)
        h2 = jnp.transpose(h, (1, 0))

        if last:
            hup = h2
        elif s == 0:
            hup = _dotb(h2, up_ref[:])
        else:
            t2 = _dotb(h2, upa_ref[:])
            hup = _dotb(t2, upb_ref[:])

        conv = _conv3x3(hup, w_ref[:], b_ref[:], xm_ref[:])
        hb = (1.0 - _RESR) * hup + _RESR * conv
        newres = chunk - hb
        out_ref[:] = newres
        lp = jnp.sum(newres * newres)
        loss_ref[:, :] = jnp.broadcast_to(lp, (8, 128))
        if last:
            fhat_ref[:] = f_ref[:] - newres

    return body


def _scale_call(s, pn, fres, emb, hen, wc, bias, xmask, f2=None):
    last = (s == _SN - 1)
    img_blk = pl.BlockSpec((_C, _P_FULL), lambda i: (i, 0))
    full = lambda shp: pl.BlockSpec(shp, lambda i: tuple(0 for _ in shp))
    in_specs = [img_blk]
    inputs = [fres]
    if last:
        in_specs.append(img_blk)
        inputs.append(f2)
    in_specs += [full((_VOCAB, _C)), full((1, _VOCAB)),
                 full((3, 3, _C, _C)), full((_C, 1)), full((8, _P_FULL))]
    inputs += [emb, hen, wc, bias, xmask]
    if s == 0:
        in_specs += [full(_DOWN1.shape), full(_UP1.shape)]
        inputs += [jnp.asarray(_DOWN1), jnp.asarray(_UP1)]
    elif not last:
        for m in (_DOWN_A[pn], _DOWN_B[pn], _UP_A[pn], _UP_B[pn]):
            in_specs.append(full(m.shape))
            inputs.append(jnp.asarray(m))

    out_shapes = [jax.ShapeDtypeStruct((_B * _C, _P_FULL), _F32),
                  jax.ShapeDtypeStruct((_B * 8, 128), _F32)]
    out_specs = [img_blk, pl.BlockSpec((8, 128), lambda i: (i, 0))]
    if last:
        out_shapes.append(jax.ShapeDtypeStruct((_B * _C, _P_FULL), _F32))
        out_specs.append(img_blk)

    return pl.pallas_call(
        _make_body(s, pn, last),
        grid=(_B,),
        in_specs=in_specs,
        out_specs=out_specs,
        out_shape=out_shapes,
        compiler_params=pltpu.CompilerParams(
            dimension_semantics=("parallel",)),
    )(*inputs)


@jax.jit
def kernel(f, emb, conv_w, conv_b):
    f = f.astype(_F32)
    f2 = f.reshape(_B * _C, _P_FULL)
    emb = emb.astype(_F32)
    hen = (0.5 * jnp.sum(emb * emb, axis=1)).reshape(1, _VOCAB)
    ksel = np.asarray(_KSEL)
    wc_all = jnp.transpose(conv_w, (0, 3, 4, 1, 2)).astype(_F32)[ksel]
    bc_all = conv_b.astype(_F32)[ksel]
    xmask = jnp.asarray(_XMASK)

    fres = f2
    loss_parts = []
    fhat2 = None
    for s, pn in enumerate(_PATCH):
        res = _scale_call(s, pn, fres, emb, hen, wc_all[s],
                          bc_all[s].reshape(_C, 1), xmask,
                          f2 if s == _SN - 1 else None)
        if s == _SN - 1:
            fres, lp, fhat2 = res
        else:
            fres, lp = res
        loss_parts.append(jnp.sum(lp) / (8.0 * 128.0))

    scale = (1.0 + _BETA) / (_SN * _B * _C * _P_FULL)
    loss = scale * sum(loss_parts)
    f_hat = fhat2.reshape(_B, _C, _HW, _HW)
    return f_hat, loss


# speedup vs baseline: 1.0222x; 1.0222x over previous
import jax
import jax.numpy as jnp
import numpy as np
from jax import lax
from jax.experimental import pallas as pl
from jax.experimental.pallas import tpu as pltpu

_VOCAB = 4096
_VT = 1024
_NV = _VOCAB // _VT
_C = 32
_B = 64
_HW = 16
_P_FULL = _HW * _HW
_PATCH = (1, 2, 3, 4, 5, 6, 8, 10, 13, 16)
_SN = len(_PATCH)
_BETA = 0.25
_RESR = 0.5
_NSHARE = 4

_TICKS = np.linspace(1.0 / 3.0 / _NSHARE, 1.0 - 1.0 / 3.0 / _NSHARE, _NSHARE)
_KSEL = [int(np.argmin(np.abs(_TICKS - i / (_SN - 1)))) for i in range(_SN)]

_PPAD = {pn: ((pn * pn + 7) // 8) * 8 for pn in _PATCH}


def _area_mat(in_size, out_size):
    W = np.zeros((out_size, in_size), dtype=np.float32)
    for i in range(out_size):
        s = (i * in_size) // out_size
        e = -((-(i + 1) * in_size) // out_size)
        W[i, s:e] = 1.0 / (e - s)
    return W


def _cubic_w(t, a=-0.75):
    at = abs(t)
    if at <= 1.0:
        return (a + 2.0) * at ** 3 - (a + 3.0) * at ** 2 + 1.0
    elif at < 2.0:
        return a * at ** 3 - 5.0 * a * at ** 2 + 8.0 * a * at - 4.0 * a
    return 0.0


def _bicubic_mat(in_size, out_size):
    W = np.zeros((out_size, in_size), dtype=np.float64)
    scale = in_size / out_size
    for i in range(out_size):
        x = (i + 0.5) * scale - 0.5
        x0 = int(np.floor(x))
        for t in range(x0 - 1, x0 + 3):
            w = _cubic_w(x - t)
            W[i, min(max(t, 0), in_size - 1)] += w
    return W.astype(np.float32)


import ml_dtypes


def _rb(a):
    return np.asarray(a, dtype=ml_dtypes.bfloat16).astype(np.float32)


def _down_mats(pn):
    M = _rb(_area_mat(_HW, pn))
    A = np.zeros((_P_FULL, _HW * pn), dtype=np.float32)
    for w in range(_HW):
        A[w::_HW, w * pn:(w + 1) * pn] = M.T
    Bm = np.zeros((_HW * pn, _PPAD[pn]), dtype=np.float32)
    for o in range(pn):
        Bm[o::pn, o * pn:(o + 1) * pn] = M.T
    return A, Bm


def _up_mats(pn):
    M = _rb(_bicubic_mat(pn, _HW))
    A = np.zeros((_PPAD[pn], pn * _HW), dtype=np.float32)
    for w in range(pn):
        A[w:pn * pn:pn, w * _HW:(w + 1) * _HW] = M.T
    Bm = np.zeros((pn * _HW, _P_FULL), dtype=np.float32)
    for o in range(_HW):
        Bm[o::_HW, o * _HW:(o + 1) * _HW] = M.T
    return A, Bm


def _kron1_down():
    M = _rb(_area_mat(_HW, 1))
    K = np.outer(M[0], M[0]).reshape(1, _P_FULL)
    Kp = np.zeros((_P_FULL, _PPAD[1]), dtype=np.float32)
    Kp[:, 0] = _rb(K[0])
    return Kp


def _kron1_up():
    M = _rb(_bicubic_mat(1, _HW))
    K = np.outer(M[:, 0], M[:, 0]).reshape(_P_FULL)
    Up = np.zeros((_PPAD[1], _P_FULL), dtype=np.float32)
    Up[0, :] = _rb(K)
    return Up


_DOWN_A = {}
_DOWN_B = {}
_UP_A = {}
_UP_B = {}
for _pn in _PATCH[1:-1]:
    _DOWN_A[_pn], _DOWN_B[_pn] = _down_mats(_pn)
    _UP_A[_pn], _UP_B[_pn] = _up_mats(_pn)
_DOWN1 = _kron1_down()
_UP1 = _kron1_up()

_XMASK = np.ones((8, _P_FULL), dtype=np.float32)
_XMASK[0, 0::16] = 0.0
_XMASK[1, 15::16] = 0.0

_F32 = jnp.float32
_HI = lax.Precision.HIGHEST


def _dot(a, b, dims):
    return lax.dot_general(a, b, (dims, ((), ())), precision=_HI,
                           preferred_element_type=_F32)


_BF16 = jnp.bfloat16


def _dotb(a, b, dims=((1,), (0,))):
    return lax.dot_general(a, b, (dims, ((), ())),
                           preferred_element_type=_F32)


def _quantize(rows, emb, hen):
    n = rows.shape[0]
    best = None
    bidx = None
    for v in range(_NV):
        ev = emb[v * _VT:(v + 1) * _VT, :]
        s = _dotb(rows, ev, ((1,), (1,))) - hen[:, v * _VT:(v + 1) * _VT]
        m = jnp.max(s, axis=1, keepdims=True)
        iota = lax.broadcasted_iota(jnp.int32, (n, _VT), 1)
        iv = jnp.min(jnp.where(s == m, iota, _VT), axis=1, keepdims=True)
        iv = iv + v * _VT
        if v == 0:
            best, bidx = m, iv
        else:
            bidx = jnp.where(m > best, iv, bidx)
            best = jnp.maximum(m, best)
    h = jnp.zeros((n, _C), _F32)
    for v in range(_NV):
        ev = emb[v * _VT:(v + 1) * _VT, :]
        iota = lax.broadcasted_iota(jnp.int32, (n, _VT), 1)
        oh = (iota == bidx - v * _VT).astype(_F32)
        h = h + _dot(oh, ev, ((1,), (0,)))
    return h


def _shift_lanes(x, o):
    if o == 0:
        return x
    z = jnp.zeros((_C, abs(o)), _F32)
    if o > 0:
        return jnp.concatenate([x[:, o:], z], axis=1)
    return jnp.concatenate([z, x[:, :o]], axis=1)


def _conv3x3(hup, w, b, xmask):
    acc = None
    for ky in range(3):
        for kx in range(3):
            o = 16 * (ky - 1) + (kx - 1)
            sh = _shift_lanes(hup, o)
            if kx == 0:
                sh = sh * xmask[0:1, :]
            elif kx == 2:
                sh = sh * xmask[1:2, :]
            t = _dotb(w[ky, kx], sh)
            acc = t if acc is None else acc + t
    return acc + b


def _make_body(s, pn, last):
    pp = _PPAD[pn]

    def body(*refs):
        if last:
            (fres_ref, f_ref, emb_ref, hen_ref, w_ref, b_ref, xm_ref,
             out_ref, loss_ref, fhat_ref) = refs
        elif s == 0:
            (fres_ref, emb_ref, hen_ref, w_ref, b_ref, xm_ref,
             down_ref, up_ref, out_ref, loss_ref) = refs
        else:
            (fres_ref, emb_ref, hen_ref, w_ref, b_ref, xm_ref,
             downa_ref, downb_ref, upa_ref, upb_ref,
             out_ref, loss_ref) = refs
        chunk = fres_ref[:]
        emb = emb_ref[:]
        hen = hen_ref[:]

        if last:
            rows = jnp.transpose(chunk, (1, 0))
        elif s == 0:
            res2 = _dotb(chunk, down_ref[:])
            rows = jnp.transpose(res2, (1, 0))
        else:
            t1 = _dotb(chunk, downa_ref[:])
            res2 = _dotb(t1, downb_ref[:])
            rows = jnp.transpose(res2, (1, 0))

        h = _quantize(rows, emb, hen)
        h2 = jnp.transpose(h, (1, 0))

        if last:
            hup = h2
        elif s == 0:
            hup = _dotb(h2, up_ref[:])
        else:
            t2 = _dotb(h2, upa_ref[:])
            hup = _dotb(t2, upb_ref[:])

        conv = _conv3x3(hup, w_ref[:], b_ref[:], xm_ref[:])
        hb = (1.0 - _RESR) * hup + _RESR * conv
        newres = chunk - hb
        out_ref[:] = newres
        lp = jnp.sum(newres * newres)
        loss_ref[:, :] = jnp.broadcast_to(lp, (8, 128))
        if last:
            fhat_ref[:] = f_ref[:] - newres

    return body


def _scale_call(s, pn, fres, emb, hen, wc, bias, xmask, f2=None):
    last = (s == _SN - 1)
    img_blk = pl.BlockSpec((_C, _P_FULL), lambda i: (i, 0))
    full = lambda shp: pl.BlockSpec(shp, lambda i: tuple(0 for _ in shp))
    in_specs = [img_blk]
    inputs = [fres]
    if last:
        in_specs.append(img_blk)
        inputs.append(f2)
    in_specs += [full((_VOCAB, _C)), full((1, _VOCAB)),
                 full((3, 3, _C, _C)), full((_C, 1)), full((8, _P_FULL))]
    inputs += [emb, hen, wc, bias, xmask]
    if s == 0:
        in_specs += [full(_DOWN1.shape), full(_UP1.shape)]
        inputs += [jnp.asarray(_DOWN1), jnp.asarray(_UP1)]
    elif not last:
        for m in (_DOWN_A[pn], _DOWN_B[pn], _UP_A[pn], _UP_B[pn]):
            in_specs.append(full(m.shape))
            inputs.append(jnp.asarray(m))

    out_shapes = [jax.ShapeDtypeStruct((_B * _C, _P_FULL), _F32),
                  jax.ShapeDtypeStruct((_B * 8, 128), _F32)]
    out_specs = [img_blk, pl.BlockSpec((8, 128), lambda i: (i, 0))]
    if last:
        out_shapes.append(jax.ShapeDtypeStruct((_B * _C, _P_FULL), _F32))
        out_specs.append(img_blk)

    return pl.pallas_call(
        _make_body(s, pn, last),
        grid=(_B,),
        in_specs=in_specs,
        out_specs=out_specs,
        out_shape=out_shapes,
        compiler_params=pltpu.CompilerParams(
            dimension_semantics=("parallel",)),
    )(*inputs)


@jax.jit
def kernel(f, emb, conv_w, conv_b):
    f = f.astype(_F32)
    f2 = f.reshape(_B * _C, _P_FULL)
    emb = emb.astype(_F32)
    hen = (0.5 * jnp.sum(emb * emb, axis=1)).reshape(1, _VOCAB)
    ksel = np.asarray(_KSEL)
    wc_all = jnp.transpose(conv_w, (0, 3, 4, 1, 2)).astype(_F32)[ksel]
    bc_all = conv_b.astype(_F32)[ksel]
    xmask = jnp.asarray(_XMASK)

    fres = f2
    loss_parts = []
    fhat2 = None
    for s, pn in enumerate(_PATCH):
        res = _scale_call(s, pn, fres, emb, hen, wc_all[s],
                          bc_all[s].reshape(_C, 1), xmask,
                          f2 if s == _SN - 1 else None)
        if s == _SN - 1:
            fres, lp, fhat2 = res
        else:
            fres, lp = res
        loss_parts.append(jnp.sum(lp) / (8.0 * 128.0))

    scale = (1.0 + _BETA) / (_SN * _B * _C * _P_FULL)
    loss = scale * sum(loss_parts)
    f_hat = fhat2.reshape(_B, _C, _HW, _HW)
    return f_hat, loss
